# baseline (speedup 1.0000x reference)
import jax
import jax.numpy as jnp
import numpy as np
from jax import lax
from jax.experimental import pallas as pl
from jax.experimental.pallas import tpu as pltpu
from jax.experimental.pallas import tpu_sc as plsc

B = 64
N = 32768
TOTAL = B * N
NUM_BANNED = 4096
L = 16
NW = 32
NS = 16
CHUNK = TOTAL // NW
ROWS_PER_W = B // NW
MIN_VAL = float(np.finfo(np.float32).min)
BIG = 2**30


def _sc_body(q_hbm, banned_hbm, vals_hbm, idx_hbm,
             q_v, banned_v, res_val_v, res_idx_v):
    c = lax.axis_index("c")
    s = lax.axis_index("s")
    wid = c * NS + s
    base = wid * CHUNK

    pltpu.sync_copy(q_hbm.at[pl.ds(base, CHUNK)], q_v)
    pltpu.sync_copy(banned_hbm, banned_v)

    lane = lax.broadcasted_iota(jnp.int32, (L,), 0)
    minvec = jnp.full((L,), MIN_VAL, dtype=jnp.float32)

    @plsc.parallel_loop(0, NUM_BANNED, L, unroll=4)
    def _(i):
        idxv = banned_v[pl.ds(i, L)]
        loc = idxv - base
        msk = (loc >= 0) & (loc < CHUNK)
        loc_safe = jnp.where(msk, loc, 0)
        plsc.store_scatter(q_v, [loc_safe], minvec, mask=msk)

    U = 8
    Q = 4
    res_val = jnp.zeros((L,), jnp.float32)
    res_idx = jnp.zeros((L,), jnp.int32)
    for r in range(ROWS_PER_W):
        m0 = tuple(jnp.full((L,), -jnp.inf, dtype=jnp.float32)
                   for _ in range(U))
        i0 = tuple(jnp.zeros((L,), jnp.int32) for _ in range(U))
        step0 = jnp.zeros((L,), jnp.int32)

        @plsc.parallel_loop(0, N, U * Q * L, unroll=2, carry=(m0, i0, step0))
        def red_step(off, carry):
            ms, mis, ivec = carry
            ms, mis = list(ms), list(mis)
            for u in range(U):
                o = off + u * Q * L
                w01 = jnp.maximum(q_v[pl.ds(r * N + o, L)],
                                  q_v[pl.ds(r * N + o + L, L)])
                w23 = jnp.maximum(q_v[pl.ds(r * N + o + 2 * L, L)],
                                  q_v[pl.ds(r * N + o + 3 * L, L)])
                w = jnp.maximum(w01, w23)
                take = w > ms[u]
                ms[u] = jnp.where(take, w, ms[u])
                mis[u] = jnp.where(take, ivec, mis[u])
            return tuple(ms), tuple(mis), ivec + 1

        ms, mis, _ = red_step
        qds = [mis[u] * U + u for u in range(U)]
        M, QD = ms[0], qds[0]
        for u in range(1, U):
            better = (ms[u] > M) | ((ms[u] == M) & (qds[u] < QD))
            M = jnp.where(better, ms[u], M)
            QD = jnp.where(better, qds[u], QD)
        m_star = jnp.max(M)
        qd_star = jnp.min(jnp.where(M == m_star, QD, BIG))
        qbase = qd_star * (Q * L)
        i_star = jnp.full((L,), BIG, jnp.int32)
        for j in range(Q):
            vj = q_v[pl.ds(r * N + qbase + j * L, L)]
            ej = jnp.where(vj == m_star, qbase + j * L + lane, BIG)
            i_star = jnp.minimum(i_star, ej)
        i_star = jnp.min(i_star)
        res_val = jnp.where(lane == r, m_star, res_val)
        res_idx = jnp.where(lane == r, i_star, res_idx)

    res_val_v[...] = res_val
    res_idx_v[...] = res_idx
    pltpu.sync_copy(res_val_v, vals_hbm.at[wid])
    pltpu.sync_copy(res_idx_v, idx_hbm.at[wid])


def kernel(q_values, banned):
    banned32 = banned.astype(jnp.int32)
    mesh = plsc.VectorSubcoreMesh(core_axis_name="c", subcore_axis_name="s")
    vals_p, idx_p = pl.kernel(
        _sc_body,
        mesh=mesh,
        out_type=[
            jax.ShapeDtypeStruct((NW, L), jnp.float32),
            jax.ShapeDtypeStruct((NW, L), jnp.int32),
        ],
        scratch_types=[
            pltpu.VMEM((CHUNK,), jnp.float32),
            pltpu.VMEM((NUM_BANNED,), jnp.int32),
            pltpu.VMEM((L,), jnp.float32),
            pltpu.VMEM((L,), jnp.int32),
        ],
        compiler_params=pltpu.CompilerParams(needs_layout_passes=False),
    )(q_values, banned32)
    values = vals_p[:, :ROWS_PER_W].reshape(B, 1)
    indices = idx_p[:, :ROWS_PER_W].reshape(B, 1)
    return values, indices


# speedup vs baseline: 1.0135x; 1.0135x over previous
import jax
import jax.numpy as jnp
import numpy as np
from jax import lax
from jax.experimental import pallas as pl
from jax.experimental.pallas import tpu as pltpu
from jax.experimental.pallas import tpu_sc as plsc

B = 64
N = 32768
TOTAL = B * N
NUM_BANNED = 4096
L = 16
NW = 32
NS = 16
CHUNK = TOTAL // NW
ROWS_PER_W = B // NW
MIN_VAL = float(np.finfo(np.float32).min)
BIG = 2**30


def _sc_body(q_hbm, banned_hbm, vals_hbm, idx_hbm,
             q_v, banned_v, res_val_v, res_idx_v):
    c = lax.axis_index("c")
    s = lax.axis_index("s")
    wid = c * NS + s
    base = wid * CHUNK

    pltpu.sync_copy(q_hbm.at[pl.ds(base, CHUNK)], q_v)
    pltpu.sync_copy(banned_hbm, banned_v)

    lane = lax.broadcasted_iota(jnp.int32, (L,), 0)
    minvec = jnp.full((L,), MIN_VAL, dtype=jnp.float32)

    @plsc.parallel_loop(0, NUM_BANNED, L, unroll=4)
    def _(i):
        idxv = banned_v[pl.ds(i, L)]
        loc = idxv - base
        msk = (loc >= 0) & (loc < CHUNK)
        loc_safe = jnp.where(msk, loc, 0)
        plsc.store_scatter(q_v, [loc_safe], minvec, mask=msk)

    U = 8
    Q = 4
    res_val = jnp.zeros((L,), jnp.float32)
    res_idx = jnp.zeros((L,), jnp.int32)
    for r in range(ROWS_PER_W):
        m0 = tuple(jnp.full((L,), -jnp.inf, dtype=jnp.float32)
                   for _ in range(U))
        i0 = tuple(jnp.zeros((L,), jnp.int32) for _ in range(U))
        step0 = jnp.zeros((L,), jnp.int32)

        @plsc.parallel_loop(0, N, U * Q * L, carry=(m0, i0, step0))
        def red_step(off, carry):
            ms, mis, ivec = carry
            ms, mis = list(ms), list(mis)
            for u in range(U):
                o = off + u * Q * L
                w01 = jnp.maximum(q_v[pl.ds(r * N + o, L)],
                                  q_v[pl.ds(r * N + o + L, L)])
                w23 = jnp.maximum(q_v[pl.ds(r * N + o + 2 * L, L)],
                                  q_v[pl.ds(r * N + o + 3 * L, L)])
                w = jnp.maximum(w01, w23)
                take = w > ms[u]
                ms[u] = jnp.where(take, w, ms[u])
                mis[u] = jnp.where(take, ivec, mis[u])
            return tuple(ms), tuple(mis), ivec + 1

        ms, mis, _ = red_step
        qds = [mis[u] * U + u for u in range(U)]
        M, QD = ms[0], qds[0]
        for u in range(1, U):
            better = (ms[u] > M) | ((ms[u] == M) & (qds[u] < QD))
            M = jnp.where(better, ms[u], M)
            QD = jnp.where(better, qds[u], QD)
        m_star = jnp.max(M)
        qd_star = jnp.min(jnp.where(M == m_star, QD, BIG))
        qbase = qd_star * (Q * L)
        i_star = jnp.full((L,), BIG, jnp.int32)
        for j in range(Q):
            vj = q_v[pl.ds(r * N + qbase + j * L, L)]
            ej = jnp.where(vj == m_star, qbase + j * L + lane, BIG)
            i_star = jnp.minimum(i_star, ej)
        i_star = jnp.min(i_star)
        res_val = jnp.where(lane == r, m_star, res_val)
        res_idx = jnp.where(lane == r, i_star, res_idx)

    res_val_v[...] = res_val
    res_idx_v[...] = res_idx
    pltpu.sync_copy(res_val_v, vals_hbm.at[wid])
    pltpu.sync_copy(res_idx_v, idx_hbm.at[wid])


def kernel(q_values, banned):
    banned32 = banned.astype(jnp.int32)
    mesh = plsc.VectorSubcoreMesh(core_axis_name="c", subcore_axis_name="s")
    vals_p, idx_p = pl.kernel(
        _sc_body,
        mesh=mesh,
        out_type=[
            jax.ShapeDtypeStruct((NW, L), jnp.float32),
            jax.ShapeDtypeStruct((NW, L), jnp.int32),
        ],
        scratch_types=[
            pltpu.VMEM((CHUNK,), jnp.float32),
            pltpu.VMEM((NUM_BANNED,), jnp.int32),
            pltpu.VMEM((L,), jnp.float32),
            pltpu.VMEM((L,), jnp.int32),
        ],
        compiler_params=pltpu.CompilerParams(needs_layout_passes=False),
    )(q_values, banned32)
    values = vals_p[:, :ROWS_PER_W].reshape(B, 1)
    indices = idx_p[:, :ROWS_PER_W].reshape(B, 1)
    return values, indices


# speedup vs baseline: 1.0341x; 1.0203x over previous
import jax
import jax.numpy as jnp
import numpy as np
from jax import lax
from jax.experimental import pallas as pl
from jax.experimental.pallas import tpu as pltpu
from jax.experimental.pallas import tpu_sc as plsc

B = 64
N = 32768
TOTAL = B * N
NUM_BANNED = 4096
L = 16
NW = 32
NS = 16
CHUNK = TOTAL // NW
ROWS_PER_W = B // NW
MIN_VAL = float(np.finfo(np.float32).min)
BIG = 2**30


def _sc_body(q_hbm, banned_hbm, vals_hbm, idx_hbm,
             q_v, banned_v, res_val_v, res_idx_v, semq):
    c = lax.axis_index("c")
    s = lax.axis_index("s")
    wid = c * NS + s
    base = wid * CHUNK

    cpq = pltpu.async_copy(q_hbm.at[pl.ds(base, CHUNK)], q_v, semq)
    pltpu.sync_copy(banned_hbm, banned_v)
    cpq.wait()

    lane = lax.broadcasted_iota(jnp.int32, (L,), 0)
    minvec = jnp.full((L,), MIN_VAL, dtype=jnp.float32)

    @plsc.parallel_loop(0, NUM_BANNED, L, unroll=4)
    def _(i):
        idxv = banned_v[pl.ds(i, L)]
        loc = idxv - base
        msk = (loc >= 0) & (loc < CHUNK)
        loc_safe = jnp.where(msk, loc, 0)
        plsc.store_scatter(q_v, [loc_safe], minvec, mask=msk)

    U = 8
    Q = 4
    res_val = jnp.zeros((L,), jnp.float32)
    res_idx = jnp.zeros((L,), jnp.int32)
    for r in range(ROWS_PER_W):
        m0 = tuple(jnp.full((L,), -jnp.inf, dtype=jnp.float32)
                   for _ in range(U))
        i0 = tuple(jnp.zeros((L,), jnp.int32) for _ in range(U))
        step0 = jnp.zeros((L,), jnp.int32)

        @plsc.parallel_loop(0, N, U * Q * L, carry=(m0, i0, step0))
        def red_step(off, carry):
            ms, mis, ivec = carry
            ms, mis = list(ms), list(mis)
            for u in range(U):
                o = off + u * Q * L
                w01 = jnp.maximum(q_v[pl.ds(r * N + o, L)],
                                  q_v[pl.ds(r * N + o + L, L)])
                w23 = jnp.maximum(q_v[pl.ds(r * N + o + 2 * L, L)],
                                  q_v[pl.ds(r * N + o + 3 * L, L)])
                w = jnp.maximum(w01, w23)
                take = w > ms[u]
                ms[u] = jnp.where(take, w, ms[u])
                mis[u] = jnp.where(take, ivec, mis[u])
            return tuple(ms), tuple(mis), ivec + 1

        ms, mis, _ = red_step
        qds = [mis[u] * U + u for u in range(U)]
        M, QD = ms[0], qds[0]
        for u in range(1, U):
            better = (ms[u] > M) | ((ms[u] == M) & (qds[u] < QD))
            M = jnp.where(better, ms[u], M)
            QD = jnp.where(better, qds[u], QD)
        m_star = jnp.max(M)
        qd_star = jnp.min(jnp.where(M == m_star, QD, BIG))
        qbase = qd_star * (Q * L)
        i_star = jnp.full((L,), BIG, jnp.int32)
        for j in range(Q):
            vj = q_v[pl.ds(r * N + qbase + j * L, L)]
            ej = jnp.where(vj == m_star, qbase + j * L + lane, BIG)
            i_star = jnp.minimum(i_star, ej)
        i_star = jnp.min(i_star)
        res_val = jnp.where(lane == r, m_star, res_val)
        res_idx = jnp.where(lane == r, i_star, res_idx)

    res_val_v[...] = res_val
    res_idx_v[...] = res_idx
    pltpu.sync_copy(res_val_v, vals_hbm.at[wid])
    pltpu.sync_copy(res_idx_v, idx_hbm.at[wid])


def kernel(q_values, banned):
    banned32 = banned.astype(jnp.int32)
    mesh = plsc.VectorSubcoreMesh(core_axis_name="c", subcore_axis_name="s")
    vals_p, idx_p = pl.kernel(
        _sc_body,
        mesh=mesh,
        out_type=[
            jax.ShapeDtypeStruct((NW, L), jnp.float32),
            jax.ShapeDtypeStruct((NW, L), jnp.int32),
        ],
        scratch_types=[
            pltpu.VMEM((CHUNK,), jnp.float32),
            pltpu.VMEM((NUM_BANNED,), jnp.int32),
            pltpu.VMEM((L,), jnp.float32),
            pltpu.VMEM((L,), jnp.int32),
            pltpu.SemaphoreType.DMA,
        ],
        compiler_params=pltpu.CompilerParams(needs_layout_passes=False),
    )(q_values, banned32)
    values = vals_p[:, :ROWS_PER_W].reshape(B, 1)
    indices = idx_p[:, :ROWS_PER_W].reshape(B, 1)
    return values, indices


# speedup vs baseline: 1.0371x; 1.0029x over previous
import jax
import jax.numpy as jnp
import numpy as np
from jax import lax
from jax.experimental import pallas as pl
from jax.experimental.pallas import tpu as pltpu
from jax.experimental.pallas import tpu_sc as plsc

B = 64
N = 32768
TOTAL = B * N
NUM_BANNED = 4096
L = 16
NW = 32
NS = 16
CHUNK = TOTAL // NW
ROWS_PER_W = B // NW
MIN_VAL = float(np.finfo(np.float32).min)
BIG = 2**30


def _sc_body(q_hbm, banned_hbm, vals_hbm, idx_hbm,
             q_v, banned_v, res_val_v, res_idx_v, semq):
    c = lax.axis_index("c")
    s = lax.axis_index("s")
    wid = c * NS + s
    base = wid * CHUNK

    cpq = pltpu.async_copy(q_hbm.at[pl.ds(base, CHUNK)], q_v, semq)
    pltpu.sync_copy(banned_hbm, banned_v)
    cpq.wait()

    lane = lax.broadcasted_iota(jnp.int32, (L,), 0)
    minvec = jnp.full((L,), MIN_VAL, dtype=jnp.float32)

    @plsc.parallel_loop(0, NUM_BANNED, L, unroll=4)
    def _(i):
        idxv = banned_v[pl.ds(i, L)]
        loc = idxv - base
        msk = (loc >= 0) & (loc < CHUNK)
        loc_safe = jnp.where(msk, loc, 0)
        plsc.store_scatter(q_v, [loc_safe], minvec, mask=msk)

    U = 8
    Q = 4
    res_val = jnp.zeros((L,), jnp.float32)
    res_idx = jnp.zeros((L,), jnp.int32)
    for r in range(ROWS_PER_W):
        m0 = tuple(jnp.full((L,), -jnp.inf, dtype=jnp.float32)
                   for _ in range(U))
        i0 = tuple(jnp.zeros((L,), jnp.int32) for _ in range(U))
        step0 = jnp.zeros((L,), jnp.int32)

        @plsc.parallel_loop(0, N, U * Q * L, carry=(m0, i0, step0))
        def red_step(off, carry):
            ms, mis, ivec = carry
            ms, mis = list(ms), list(mis)
            for u in range(U):
                o = off + u * Q * L
                w01 = jnp.maximum(q_v[pl.ds(r * N + o, L)],
                                  q_v[pl.ds(r * N + o + L, L)])
                w23 = jnp.maximum(q_v[pl.ds(r * N + o + 2 * L, L)],
                                  q_v[pl.ds(r * N + o + 3 * L, L)])
                w = jnp.maximum(w01, w23)
                take = w > ms[u]
                ms[u] = jnp.where(take, w, ms[u])
                mis[u] = jnp.where(take, ivec, mis[u])
            return tuple(ms), tuple(mis), ivec + 1

        ms, mis, _ = red_step
        qds = [mis[u] * U + u for u in range(U)]
        M, QD = ms[0], qds[0]
        for u in range(1, U):
            better = (ms[u] > M) | ((ms[u] == M) & (qds[u] < QD))
            M = jnp.where(better, ms[u], M)
            QD = jnp.where(better, qds[u], QD)
        m_star = jnp.max(M)
        qd_star = jnp.min(jnp.where(M == m_star, QD, BIG))
        qbase = qd_star * (Q * L)
        i_star = jnp.full((L,), BIG, jnp.int32)
        for j in range(Q):
            vj = q_v[pl.ds(r * N + qbase + j * L, L)]
            ej = jnp.where(vj == m_star, qbase + j * L + lane, BIG)
            i_star = jnp.minimum(i_star, ej)
        i_star = jnp.min(i_star)
        res_val = jnp.where(lane == r, m_star, res_val)
        res_idx = jnp.where(lane == r, i_star, res_idx)

    res_val_v[...] = res_val
    res_idx_v[...] = res_idx
    cpv = pltpu.async_copy(res_val_v, vals_hbm.at[wid], semq)
    pltpu.sync_copy(res_idx_v, idx_hbm.at[wid])
    cpv.wait()


def kernel(q_values, banned):
    banned32 = banned.astype(jnp.int32)
    mesh = plsc.VectorSubcoreMesh(core_axis_name="c", subcore_axis_name="s")
    vals_p, idx_p = pl.kernel(
        _sc_body,
        mesh=mesh,
        out_type=[
            jax.ShapeDtypeStruct((NW, L), jnp.float32),
            jax.ShapeDtypeStruct((NW, L), jnp.int32),
        ],
        scratch_types=[
            pltpu.VMEM((CHUNK,), jnp.float32),
            pltpu.VMEM((NUM_BANNED,), jnp.int32),
            pltpu.VMEM((L,), jnp.float32),
            pltpu.VMEM((L,), jnp.int32),
            pltpu.SemaphoreType.DMA,
        ],
        compiler_params=pltpu.CompilerParams(needs_layout_passes=False),
    )(q_values, banned32)
    values = vals_p[:, :ROWS_PER_W].reshape(B, 1)
    indices = idx_p[:, :ROWS_PER_W].reshape(B, 1)
    return values, indices
